# baseline (speedup 1.0000x reference)
import functools

import jax
import jax.numpy as jnp
from jax import lax
from jax.experimental import pallas as pl
from jax.experimental.pallas import tpu as pltpu
from jax.experimental.pallas import tpu_sc as plsc

V = 50000
K = 512
D = 256
N = 32768

NC = 2
NS = 16
NW = NC * NS
TOK_PER_TILE = N // NW
GCHUNK = 128
NCHUNK = TOK_PER_TILE // GCHUNK


def _sc_gather(doc, emb_w):
    mesh = plsc.VectorSubcoreMesh(core_axis_name="c", subcore_axis_name="s")

    @functools.partial(
        pl.kernel,
        mesh=mesh,
        out_type=jax.ShapeDtypeStruct((N, D), jnp.float32),
        scratch_types=[
            pltpu.VMEM((NCHUNK, GCHUNK), jnp.int32),
            pltpu.VMEM((GCHUNK, D), jnp.float32),
            pltpu.SemaphoreType.DMA,
        ],
    )
    def k(doc_hbm, emb_hbm, out_hbm, idx_v, rows_v, sem):
        cid = lax.axis_index("c")
        sid = lax.axis_index("s")
        wid = sid * NC + cid
        base = wid * TOK_PER_TILE

        for j in range(NCHUNK):
            pltpu.sync_copy(doc_hbm.at[pl.ds(base + j * GCHUNK, GCHUNK)],
                            idx_v.at[j])
        for j in range(NCHUNK):
            pltpu.async_copy(emb_hbm.at[idx_v.at[j]], rows_v, sem).wait()
            pltpu.sync_copy(rows_v,
                            out_hbm.at[pl.ds(base + j * GCHUNK, GCHUNK)])

    return k(doc, emb_w)


def _sc_quantize_bincount(idx_n, cw, doc, zeros_v, ones_g):
    mesh = plsc.VectorSubcoreMesh(core_axis_name="c", subcore_axis_name="s")

    @functools.partial(
        pl.kernel,
        mesh=mesh,
        out_type=(
            jax.ShapeDtypeStruct((N, D), jnp.float32),
            jax.ShapeDtypeStruct((NC, V), jnp.float32),
        ),
        scratch_types=[
            pltpu.VMEM((NCHUNK, GCHUNK), jnp.int32),
            pltpu.VMEM((NCHUNK, GCHUNK), jnp.int32),
            pltpu.VMEM((GCHUNK, D), jnp.float32),
            pltpu.VMEM((GCHUNK,), jnp.float32),
            pltpu.VMEM_SHARED((V,), jnp.float32),
            pltpu.SemaphoreType.DMA,
        ],
    )
    def k(idx_hbm, cw_hbm, doc_hbm, zeros_hbm, ones_hbm, qw_hbm, bc_hbm,
          idx_v, doc_v, rows_v, ones_v, hist_sh, sem):
        cid = lax.axis_index("c")
        sid = lax.axis_index("s")
        wid = sid * NC + cid
        base = wid * TOK_PER_TILE

        @pl.when(sid == 0)
        def _():
            pltpu.sync_copy(zeros_hbm, hist_sh)

        pltpu.sync_copy(ones_hbm, ones_v)
        for j in range(NCHUNK):
            pltpu.sync_copy(idx_hbm.at[pl.ds(base + j * GCHUNK, GCHUNK)],
                            idx_v.at[j])
            pltpu.sync_copy(doc_hbm.at[pl.ds(base + j * GCHUNK, GCHUNK)],
                            doc_v.at[j])
        plsc.subcore_barrier()

        for j in range(NCHUNK):
            pltpu.async_copy(cw_hbm.at[idx_v.at[j]], rows_v, sem).wait()
            pltpu.sync_copy(rows_v,
                            qw_hbm.at[pl.ds(base + j * GCHUNK, GCHUNK)])
            pltpu.sync_copy(ones_v, hist_sh.at[doc_v.at[j]], add=True)

        plsc.subcore_barrier()

        @pl.when(sid == 0)
        def _():
            pltpu.sync_copy(hist_sh, bc_hbm.at[cid])

    return k(idx_n, cw, doc, zeros_v, ones_g)


BN = 512
NB = N // BN


def _vq_body(e_ref, c_ref, enc_ref, idx_ref, cnt_ref, vq_ref, acc_ref, vqs_ref):
    i = pl.program_id(0)
    e = e_ref[...]
    c = c_ref[...]
    e2 = jnp.sum(e * e, axis=1, keepdims=True)
    c2 = jnp.sum(c * c, axis=1)
    cross = lax.dot_general(e, c, (((1,), (1,)), ((), ())))
    dist = e2 + c2[None, :] - 2.0 * cross
    m = jnp.min(dist, axis=1, keepdims=True)
    kiota = lax.broadcasted_iota(jnp.int32, (BN, K), 1)
    idx = jnp.min(jnp.where(dist == m, kiota, K), axis=1, keepdims=True)
    onehot = (kiota == idx).astype(jnp.float32)
    enc_ref[...] = onehot
    idx_ref[...] = idx

    @pl.when(i == 0)
    def _():
        acc_ref[...] = jnp.zeros_like(acc_ref)
        vqs_ref[0, 0] = 0.0

    acc_ref[...] += jnp.sum(onehot, axis=0, keepdims=True)
    vqs_ref[0, 0] += jnp.sum(m)

    @pl.when(i == NB - 1)
    def _():
        cnt_ref[...] = acc_ref[...]
        mloss = vqs_ref[0, 0] / (N * D)
        vq_ref[0, 0] = mloss + 0.25 * mloss


def _tc_vq(embedded, cw):
    return pl.pallas_call(
        _vq_body,
        grid=(NB,),
        in_specs=[
            pl.BlockSpec((BN, D), lambda i: (i, 0)),
            pl.BlockSpec((K, D), lambda i: (0, 0)),
        ],
        out_specs=[
            pl.BlockSpec((BN, K), lambda i: (i, 0)),
            pl.BlockSpec((BN, 1), lambda i: (i, 0)),
            pl.BlockSpec((1, K), lambda i: (0, 0)),
            pl.BlockSpec((1, 1), lambda i: (0, 0), memory_space=pltpu.SMEM),
        ],
        out_shape=[
            jax.ShapeDtypeStruct((N, K), jnp.float32),
            jax.ShapeDtypeStruct((N, 1), jnp.int32),
            jax.ShapeDtypeStruct((1, K), jnp.float32),
            jax.ShapeDtypeStruct((1, 1), jnp.float32),
        ],
        scratch_shapes=[
            pltpu.VMEM((1, K), jnp.float32),
            pltpu.SMEM((1, 1), jnp.float32),
        ],
    )(embedded, cw)


def _lts_body(c_ref, out_ref):
    c = c_ref[...]
    g = lax.dot_general(c, c, (((1,), (1,)), ((), ())))
    nrm = jnp.sum(c * c, axis=1)
    sm = jnp.sum(c, axis=1)
    d2 = (nrm[:, None] + nrm[None, :] - 2.0 * g
          + 2e-6 * (sm[:, None] - sm[None, :]) + D * 1e-12)
    dist = jnp.sqrt(jnp.maximum(d2, 0.0))
    r = lax.broadcasted_iota(jnp.int32, (K, K), 0)
    cc = lax.broadcasted_iota(jnp.int32, (K, K), 1)
    losses = jnp.where(r == cc, dist, jnp.maximum(0.0, 1.0 - dist))
    out_ref[0, 0] = jnp.sum(losses) / (K * K)


def _tc_lts(cw):
    return pl.pallas_call(
        _lts_body,
        out_specs=pl.BlockSpec(memory_space=pltpu.SMEM),
        out_shape=jax.ShapeDtypeStruct((1, 1), jnp.float32),
    )(cw)


BV = 2048
NVB = (V + BV - 1) // BV


def _logits_body(w_ref, b_ref, cnt_ref, c_ref, lg_ref, m_ref, s_ref, docu_ref,
                 mm_ref, ss_ref, docu_v):
    j = pl.program_id(0)

    @pl.when(j == 0)
    def _():
        docu_v[...] = lax.dot_general(cnt_ref[...], c_ref[...],
                                      (((1,), (0,)), ((), ()))) / N
        docu_ref[...] = docu_v[...]

    w = w_ref[...]
    docu = docu_v[...]
    lg = lax.dot_general(docu, w, (((1,), (1,)), ((), ()))) + b_ref[...]
    lg_ref[...] = lg
    viota = lax.broadcasted_iota(jnp.int32, (1, BV), 1) + j * BV
    valid = viota < V
    lgv = jnp.where(valid, lg, -jnp.inf)
    bm = jnp.max(lgv)

    @pl.when(j == 0)
    def _():
        mm_ref[0, 0] = -jnp.inf
        ss_ref[0, 0] = 0.0

    m_old = mm_ref[0, 0]
    m_new = jnp.maximum(m_old, bm)
    ssum = jnp.sum(jnp.where(valid, jnp.exp(lg - m_new), 0.0))
    ss_ref[0, 0] = ss_ref[0, 0] * jnp.exp(m_old - m_new) + ssum
    mm_ref[0, 0] = m_new

    @pl.when(j == NVB - 1)
    def _():
        m_ref[0, 0] = mm_ref[0, 0]
        s_ref[0, 0] = ss_ref[0, 0]


def _tc_logits(q2v_W, q2v_b2d, cnt, cw):
    return pl.pallas_call(
        _logits_body,
        grid=(NVB,),
        in_specs=[
            pl.BlockSpec((BV, D), lambda j: (j, 0)),
            pl.BlockSpec((1, BV), lambda j: (0, j)),
            pl.BlockSpec((1, K), lambda j: (0, 0)),
            pl.BlockSpec((K, D), lambda j: (0, 0)),
        ],
        out_specs=[
            pl.BlockSpec((1, BV), lambda j: (0, j)),
            pl.BlockSpec((1, 1), lambda j: (0, 0), memory_space=pltpu.SMEM),
            pl.BlockSpec((1, 1), lambda j: (0, 0), memory_space=pltpu.SMEM),
            pl.BlockSpec((1, D), lambda j: (0, 0)),
        ],
        out_shape=[
            jax.ShapeDtypeStruct((1, V), jnp.float32),
            jax.ShapeDtypeStruct((1, 1), jnp.float32),
            jax.ShapeDtypeStruct((1, 1), jnp.float32),
            jax.ShapeDtypeStruct((1, D), jnp.float32),
        ],
        scratch_shapes=[
            pltpu.SMEM((1, 1), jnp.float32),
            pltpu.SMEM((1, 1), jnp.float32),
            pltpu.VMEM((1, D), jnp.float32),
        ],
    )(q2v_W, q2v_b2d, cnt, cw)


def _final_body(lg_ref, m_ref, s_ref, bc_ref, out_ref):
    lg = lg_ref[...]
    smax = jnp.exp(lg - m_ref[0, 0]) / s_ref[0, 0]
    bc = jnp.sum(bc_ref[...], axis=0, keepdims=True)
    out_ref[...] = jnp.log(smax + 1e-6) * bc


def _tc_finalize(lg, m, s, bc2):
    return pl.pallas_call(
        _final_body,
        grid=(NVB,),
        in_specs=[
            pl.BlockSpec((1, BV), lambda j: (0, j)),
            pl.BlockSpec((1, 1), lambda j: (0, 0), memory_space=pltpu.SMEM),
            pl.BlockSpec((1, 1), lambda j: (0, 0), memory_space=pltpu.SMEM),
            pl.BlockSpec((NC, BV), lambda j: (0, j)),
        ],
        out_specs=pl.BlockSpec((1, BV), lambda j: (0, j)),
        out_shape=jax.ShapeDtypeStruct((1, V), jnp.float32),
    )(lg, m, s, bc2)


def kernel(input_document, emb_w, emb_concept_w, q2v_W, q2v_b):
    doc = input_document.astype(jnp.int32)
    zeros_v = jnp.zeros((V,), jnp.float32)
    ones_g = jnp.ones((GCHUNK,), jnp.float32)
    embedded = _sc_gather(doc, emb_w)
    enc, idx, cnt, vq = _tc_vq(embedded, emb_concept_w)
    qw, bc2 = _sc_quantize_bincount(idx.reshape(N), emb_concept_w, doc,
                                    zeros_v, ones_g)
    lts = _tc_lts(emb_concept_w)
    lg, m, s, docu = _tc_logits(q2v_W, q2v_b.reshape(1, V), cnt,
                                emb_concept_w)
    outs = _tc_finalize(lg, m, s, bc2)
    return (enc, qw, docu, outs, vq.reshape(()), lts.reshape(()))


# speedup vs baseline: 1.1323x; 1.1323x over previous
import functools

import jax
import jax.numpy as jnp
from jax import lax
from jax.experimental import pallas as pl
from jax.experimental.pallas import tpu as pltpu
from jax.experimental.pallas import tpu_sc as plsc

V = 50000
K = 512
D = 256
N = 32768

NC = 2
NS = 16
NW = NC * NS
TOK_PER_TILE = N // NW
GCHUNK = 128
NCHUNK = TOK_PER_TILE // GCHUNK


def _sc_gather_bincount(doc, emb_w, zeros_v, ones_g):
    mesh = plsc.VectorSubcoreMesh(core_axis_name="c", subcore_axis_name="s")

    @functools.partial(
        pl.kernel,
        mesh=mesh,
        out_type=(
            jax.ShapeDtypeStruct((N, D), jnp.float32),
            jax.ShapeDtypeStruct((NC, V), jnp.float32),
        ),
        scratch_types=[
            pltpu.VMEM((NCHUNK, GCHUNK), jnp.int32),
            pltpu.VMEM((GCHUNK, D), jnp.float32),
            pltpu.VMEM((GCHUNK,), jnp.float32),
            pltpu.VMEM_SHARED((V,), jnp.float32),
            pltpu.SemaphoreType.DMA,
        ],
    )
    def k(doc_hbm, emb_hbm, zeros_hbm, ones_hbm, out_hbm, bc_hbm,
          idx_v, rows_v, ones_v, hist_sh, sem):
        cid = lax.axis_index("c")
        sid = lax.axis_index("s")
        wid = sid * NC + cid
        base = wid * TOK_PER_TILE

        @pl.when(sid == 0)
        def _():
            pltpu.sync_copy(zeros_hbm, hist_sh)

        pltpu.sync_copy(ones_hbm, ones_v)
        for j in range(NCHUNK):
            pltpu.sync_copy(doc_hbm.at[pl.ds(base + j * GCHUNK, GCHUNK)],
                            idx_v.at[j])
        plsc.subcore_barrier()

        for j in range(NCHUNK):
            pltpu.async_copy(emb_hbm.at[idx_v.at[j]], rows_v, sem).wait()
            pltpu.sync_copy(rows_v,
                            out_hbm.at[pl.ds(base + j * GCHUNK, GCHUNK)])
            pltpu.sync_copy(ones_v, hist_sh.at[idx_v.at[j]], add=True)

        plsc.subcore_barrier()

        @pl.when(sid == 0)
        def _():
            pltpu.sync_copy(hist_sh, bc_hbm.at[cid])

    return k(doc, emb_w, zeros_v, ones_g)


BN = 512
NB = N // BN


def _vq_body(e_ref, c_ref, enc_ref, qw_ref, cnt_ref, vq_ref, acc_ref, vqs_ref):
    i = pl.program_id(0)
    e = e_ref[...]
    c = c_ref[...]
    e2 = jnp.sum(e * e, axis=1, keepdims=True)
    c2 = jnp.sum(c * c, axis=1)
    cross = lax.dot_general(e, c, (((1,), (1,)), ((), ())))
    dist = e2 + c2[None, :] - 2.0 * cross
    m = jnp.min(dist, axis=1, keepdims=True)
    kiota = lax.broadcasted_iota(jnp.int32, (BN, K), 1)
    idx = jnp.min(jnp.where(dist == m, kiota, K), axis=1, keepdims=True)
    onehot = (kiota == idx).astype(jnp.float32)
    enc_ref[...] = onehot
    qw_ref[...] = jnp.dot(onehot, c)

    @pl.when(i == 0)
    def _():
        acc_ref[...] = jnp.zeros_like(acc_ref)
        vqs_ref[0, 0] = 0.0

    acc_ref[...] += jnp.sum(onehot, axis=0, keepdims=True)
    vqs_ref[0, 0] += jnp.sum(m)

    @pl.when(i == NB - 1)
    def _():
        cnt_ref[...] = acc_ref[...]
        mloss = vqs_ref[0, 0] / (N * D)
        vq_ref[0, 0] = mloss + 0.25 * mloss


def _tc_vq(embedded, cw):
    return pl.pallas_call(
        _vq_body,
        grid=(NB,),
        in_specs=[
            pl.BlockSpec((BN, D), lambda i: (i, 0)),
            pl.BlockSpec((K, D), lambda i: (0, 0)),
        ],
        out_specs=[
            pl.BlockSpec((BN, K), lambda i: (i, 0)),
            pl.BlockSpec((BN, D), lambda i: (i, 0)),
            pl.BlockSpec((1, K), lambda i: (0, 0)),
            pl.BlockSpec((1, 1), lambda i: (0, 0), memory_space=pltpu.SMEM),
        ],
        out_shape=[
            jax.ShapeDtypeStruct((N, K), jnp.float32),
            jax.ShapeDtypeStruct((N, D), jnp.float32),
            jax.ShapeDtypeStruct((1, K), jnp.float32),
            jax.ShapeDtypeStruct((1, 1), jnp.float32),
        ],
        scratch_shapes=[
            pltpu.VMEM((1, K), jnp.float32),
            pltpu.SMEM((1, 1), jnp.float32),
        ],
    )(embedded, cw)


def _lts_body(c_ref, out_ref):
    c = c_ref[...]
    g = lax.dot_general(c, c, (((1,), (1,)), ((), ())))
    nrm = jnp.sum(c * c, axis=1)
    sm = jnp.sum(c, axis=1)
    d2 = (nrm[:, None] + nrm[None, :] - 2.0 * g
          + 2e-6 * (sm[:, None] - sm[None, :]) + D * 1e-12)
    dist = jnp.sqrt(jnp.maximum(d2, 0.0))
    r = lax.broadcasted_iota(jnp.int32, (K, K), 0)
    cc = lax.broadcasted_iota(jnp.int32, (K, K), 1)
    losses = jnp.where(r == cc, dist, jnp.maximum(0.0, 1.0 - dist))
    out_ref[0, 0] = jnp.sum(losses) / (K * K)


def _tc_lts(cw):
    return pl.pallas_call(
        _lts_body,
        out_specs=pl.BlockSpec(memory_space=pltpu.SMEM),
        out_shape=jax.ShapeDtypeStruct((1, 1), jnp.float32),
    )(cw)


BV = 2048
NVB = (V + BV - 1) // BV


def _logits_body(w_ref, b_ref, cnt_ref, c_ref, lg_ref, m_ref, s_ref, docu_ref,
                 mm_ref, ss_ref, docu_v):
    j = pl.program_id(0)

    @pl.when(j == 0)
    def _():
        docu_v[...] = lax.dot_general(cnt_ref[...], c_ref[...],
                                      (((1,), (0,)), ((), ()))) / N
        docu_ref[...] = docu_v[...]

    w = w_ref[...]
    docu = docu_v[...]
    lg = lax.dot_general(docu, w, (((1,), (1,)), ((), ()))) + b_ref[...]
    lg_ref[...] = lg
    viota = lax.broadcasted_iota(jnp.int32, (1, BV), 1) + j * BV
    valid = viota < V
    lgv = jnp.where(valid, lg, -jnp.inf)
    bm = jnp.max(lgv)

    @pl.when(j == 0)
    def _():
        mm_ref[0, 0] = -jnp.inf
        ss_ref[0, 0] = 0.0

    m_old = mm_ref[0, 0]
    m_new = jnp.maximum(m_old, bm)
    ssum = jnp.sum(jnp.where(valid, jnp.exp(lg - m_new), 0.0))
    ss_ref[0, 0] = ss_ref[0, 0] * jnp.exp(m_old - m_new) + ssum
    mm_ref[0, 0] = m_new

    @pl.when(j == NVB - 1)
    def _():
        m_ref[0, 0] = mm_ref[0, 0]
        s_ref[0, 0] = ss_ref[0, 0]


def _tc_logits(q2v_W, q2v_b2d, cnt, cw):
    return pl.pallas_call(
        _logits_body,
        grid=(NVB,),
        in_specs=[
            pl.BlockSpec((BV, D), lambda j: (j, 0)),
            pl.BlockSpec((1, BV), lambda j: (0, j)),
            pl.BlockSpec((1, K), lambda j: (0, 0)),
            pl.BlockSpec((K, D), lambda j: (0, 0)),
        ],
        out_specs=[
            pl.BlockSpec((1, BV), lambda j: (0, j)),
            pl.BlockSpec((1, 1), lambda j: (0, 0), memory_space=pltpu.SMEM),
            pl.BlockSpec((1, 1), lambda j: (0, 0), memory_space=pltpu.SMEM),
            pl.BlockSpec((1, D), lambda j: (0, 0)),
        ],
        out_shape=[
            jax.ShapeDtypeStruct((1, V), jnp.float32),
            jax.ShapeDtypeStruct((1, 1), jnp.float32),
            jax.ShapeDtypeStruct((1, 1), jnp.float32),
            jax.ShapeDtypeStruct((1, D), jnp.float32),
        ],
        scratch_shapes=[
            pltpu.SMEM((1, 1), jnp.float32),
            pltpu.SMEM((1, 1), jnp.float32),
            pltpu.VMEM((1, D), jnp.float32),
        ],
    )(q2v_W, q2v_b2d, cnt, cw)


def _final_body(lg_ref, m_ref, s_ref, bc_ref, out_ref):
    lg = lg_ref[...]
    smax = jnp.exp(lg - m_ref[0, 0]) / s_ref[0, 0]
    bc = jnp.sum(bc_ref[...], axis=0, keepdims=True)
    out_ref[...] = jnp.log(smax + 1e-6) * bc


def _tc_finalize(lg, m, s, bc2):
    return pl.pallas_call(
        _final_body,
        grid=(NVB,),
        in_specs=[
            pl.BlockSpec((1, BV), lambda j: (0, j)),
            pl.BlockSpec((1, 1), lambda j: (0, 0), memory_space=pltpu.SMEM),
            pl.BlockSpec((1, 1), lambda j: (0, 0), memory_space=pltpu.SMEM),
            pl.BlockSpec((NC, BV), lambda j: (0, j)),
        ],
        out_specs=pl.BlockSpec((1, BV), lambda j: (0, j)),
        out_shape=jax.ShapeDtypeStruct((1, V), jnp.float32),
    )(lg, m, s, bc2)


def kernel(input_document, emb_w, emb_concept_w, q2v_W, q2v_b):
    doc = input_document.astype(jnp.int32)
    zeros_v = jnp.zeros((V,), jnp.float32)
    ones_g = jnp.ones((GCHUNK,), jnp.float32)
    embedded, bc2 = _sc_gather_bincount(doc, emb_w, zeros_v, ones_g)
    enc, qw, cnt, vq = _tc_vq(embedded, emb_concept_w)
    lts = _tc_lts(emb_concept_w)
    lg, m, s, docu = _tc_logits(q2v_W, q2v_b.reshape(1, V), cnt,
                                emb_concept_w)
    outs = _tc_finalize(lg, m, s, bc2)
    return (enc, qw, docu, outs, vq.reshape(()), lts.reshape(()))


# speedup vs baseline: 1.3791x; 1.2180x over previous
import functools

import jax
import jax.numpy as jnp
from jax import lax
from jax.experimental import pallas as pl
from jax.experimental.pallas import tpu as pltpu
from jax.experimental.pallas import tpu_sc as plsc

V = 50000
K = 512
D = 256
N = 32768

NC = 2
NS = 16
NW = NC * NS
TOK_PER_TILE = N // NW
GCHUNK = 128
NCHUNK = TOK_PER_TILE // GCHUNK


def _sc_gather_bincount(doc, emb_w, zeros_v, ones_g):
    mesh = plsc.VectorSubcoreMesh(core_axis_name="c", subcore_axis_name="s")

    @functools.partial(
        pl.kernel,
        mesh=mesh,
        out_type=(
            jax.ShapeDtypeStruct((N, D), jnp.float32),
            jax.ShapeDtypeStruct((NC, V), jnp.float32),
        ),
        scratch_types=[
            pltpu.VMEM((NCHUNK, GCHUNK), jnp.int32),
            pltpu.VMEM((GCHUNK, D), jnp.float32),
            pltpu.VMEM((GCHUNK,), jnp.float32),
            pltpu.VMEM_SHARED((V,), jnp.float32),
            pltpu.SemaphoreType.DMA,
        ],
    )
    def k(doc_hbm, emb_hbm, zeros_hbm, ones_hbm, out_hbm, bc_hbm,
          idx_v, rows_v, ones_v, hist_sh, sem):
        cid = lax.axis_index("c")
        sid = lax.axis_index("s")
        wid = sid * NC + cid
        base = wid * TOK_PER_TILE

        @pl.when(sid == 0)
        def _():
            pltpu.sync_copy(zeros_hbm, hist_sh)

        pltpu.sync_copy(ones_hbm, ones_v)
        for j in range(NCHUNK):
            pltpu.sync_copy(doc_hbm.at[pl.ds(base + j * GCHUNK, GCHUNK)],
                            idx_v.at[j])
        plsc.subcore_barrier()

        for j in range(NCHUNK):
            pltpu.async_copy(emb_hbm.at[idx_v.at[j]], rows_v, sem).wait()
            pltpu.sync_copy(rows_v,
                            out_hbm.at[pl.ds(base + j * GCHUNK, GCHUNK)])
            pltpu.sync_copy(ones_v, hist_sh.at[idx_v.at[j]], add=True)

        plsc.subcore_barrier()

        @pl.when(sid == 0)
        def _():
            pltpu.sync_copy(hist_sh, bc_hbm.at[cid])

    return k(doc, emb_w, zeros_v, ones_g)


BN = 4096
NB = N // BN


def _vq_body(e_ref, c_ref, enc_ref, qw_ref, cnt_ref, vq_ref, lts_ref,
             acc_ref, vqs_ref):
    i = pl.program_id(0)
    e = e_ref[...]
    c = c_ref[...]

    @pl.when(i == 0)
    def _():
        g = lax.dot_general(c, c, (((1,), (1,)), ((), ())))
        nrm = jnp.sum(c * c, axis=1)
        sm = jnp.sum(c, axis=1)
        d2 = (nrm[:, None] + nrm[None, :] - 2.0 * g
              + 2e-6 * (sm[:, None] - sm[None, :]) + D * 1e-12)
        dist = jnp.sqrt(jnp.maximum(d2, 0.0))
        r = lax.broadcasted_iota(jnp.int32, (K, K), 0)
        cc = lax.broadcasted_iota(jnp.int32, (K, K), 1)
        losses = jnp.where(r == cc, dist, jnp.maximum(0.0, 1.0 - dist))
        lts_ref[0, 0] = jnp.sum(losses) / (K * K)
    e2 = jnp.sum(e * e, axis=1, keepdims=True)
    c2 = jnp.sum(c * c, axis=1)
    cross = lax.dot_general(e, c, (((1,), (1,)), ((), ())))
    dist = e2 + c2[None, :] - 2.0 * cross
    m = jnp.min(dist, axis=1, keepdims=True)
    kiota = lax.broadcasted_iota(jnp.int32, (BN, K), 1)
    idx = jnp.min(jnp.where(dist == m, kiota, K), axis=1, keepdims=True)
    onehot = (kiota == idx).astype(jnp.float32)
    enc_ref[...] = onehot
    qw_ref[...] = jnp.dot(onehot, c)

    @pl.when(i == 0)
    def _():
        acc_ref[...] = jnp.zeros_like(acc_ref)
        vqs_ref[0, 0] = 0.0

    acc_ref[...] += jnp.sum(onehot, axis=0, keepdims=True)
    vqs_ref[0, 0] += jnp.sum(m)

    @pl.when(i == NB - 1)
    def _():
        cnt_ref[...] = acc_ref[...]
        mloss = vqs_ref[0, 0] / (N * D)
        vq_ref[0, 0] = mloss + 0.25 * mloss


def _tc_vq(embedded, cw):
    return pl.pallas_call(
        _vq_body,
        grid=(NB,),
        in_specs=[
            pl.BlockSpec((BN, D), lambda i: (i, 0)),
            pl.BlockSpec((K, D), lambda i: (0, 0)),
        ],
        out_specs=[
            pl.BlockSpec((BN, K), lambda i: (i, 0)),
            pl.BlockSpec((BN, D), lambda i: (i, 0)),
            pl.BlockSpec((1, K), lambda i: (0, 0)),
            pl.BlockSpec((1, 1), lambda i: (0, 0), memory_space=pltpu.SMEM),
            pl.BlockSpec((1, 1), lambda i: (0, 0), memory_space=pltpu.SMEM),
        ],
        out_shape=[
            jax.ShapeDtypeStruct((N, K), jnp.float32),
            jax.ShapeDtypeStruct((N, D), jnp.float32),
            jax.ShapeDtypeStruct((1, K), jnp.float32),
            jax.ShapeDtypeStruct((1, 1), jnp.float32),
            jax.ShapeDtypeStruct((1, 1), jnp.float32),
        ],
        scratch_shapes=[
            pltpu.VMEM((1, K), jnp.float32),
            pltpu.SMEM((1, 1), jnp.float32),
        ],
    )(embedded, cw)


BV = 2048
NVB = (V + BV - 1) // BV
VPAD = NVB * BV


def _vocab_body(w_ref, b_ref, cnt_ref, c_ref, bc_ref,
                out_ref, m_ref, s_ref, docu_ref,
                mm_ref, ss_ref, docu_v, lg_scr):
    p = pl.program_id(0)
    j = pl.program_id(1)

    @pl.when((p == 0) & (j == 0))
    def _():
        docu_v[...] = lax.dot_general(cnt_ref[...], c_ref[...],
                                      (((1,), (0,)), ((), ()))) / N
        docu_ref[...] = docu_v[...]
        mm_ref[0, 0] = -jnp.inf
        ss_ref[0, 0] = 0.0

    @pl.when(p == 0)
    def _():
        w = w_ref[...]
        docu = docu_v[...]
        lg = lax.dot_general(docu, w, (((1,), (1,)), ((), ()))) + b_ref[...]
        lg_scr[0:1, pl.ds(j * BV, BV)] = lg
        viota = lax.broadcasted_iota(jnp.int32, (1, BV), 1) + j * BV
        valid = viota < V
        lgv = jnp.where(valid, lg, -jnp.inf)
        bm = jnp.max(lgv)
        m_old = mm_ref[0, 0]
        m_new = jnp.maximum(m_old, bm)
        ssum = jnp.sum(jnp.where(valid, jnp.exp(lg - m_new), 0.0))
        ss_ref[0, 0] = ss_ref[0, 0] * jnp.exp(m_old - m_new) + ssum
        mm_ref[0, 0] = m_new

        @pl.when(j == NVB - 1)
        def _():
            m_ref[0, 0] = mm_ref[0, 0]
            s_ref[0, 0] = ss_ref[0, 0]

    @pl.when(p == 1)
    def _():
        lg = lg_scr[0:1, pl.ds(j * BV, BV)]
        smax = jnp.exp(lg - mm_ref[0, 0]) / ss_ref[0, 0]
        bc = jnp.sum(bc_ref[...], axis=0, keepdims=True)
        out_ref[...] = jnp.log(smax + 1e-6) * bc


def _tc_vocab(q2v_W, q2v_b2d, cnt, cw, bc2):
    return pl.pallas_call(
        _vocab_body,
        grid=(2, NVB),
        in_specs=[
            pl.BlockSpec((BV, D), lambda p, j: ((1 - p) * j, 0)),
            pl.BlockSpec((1, BV), lambda p, j: (0, (1 - p) * j)),
            pl.BlockSpec((1, K), lambda p, j: (0, 0)),
            pl.BlockSpec((K, D), lambda p, j: (0, 0)),
            pl.BlockSpec((NC, BV), lambda p, j: (0, p * j)),
        ],
        out_specs=[
            pl.BlockSpec((1, BV), lambda p, j: (0, p * j)),
            pl.BlockSpec((1, 1), lambda p, j: (0, 0), memory_space=pltpu.SMEM),
            pl.BlockSpec((1, 1), lambda p, j: (0, 0), memory_space=pltpu.SMEM),
            pl.BlockSpec((1, D), lambda p, j: (0, 0)),
        ],
        out_shape=[
            jax.ShapeDtypeStruct((1, V), jnp.float32),
            jax.ShapeDtypeStruct((1, 1), jnp.float32),
            jax.ShapeDtypeStruct((1, 1), jnp.float32),
            jax.ShapeDtypeStruct((1, D), jnp.float32),
        ],
        scratch_shapes=[
            pltpu.SMEM((1, 1), jnp.float32),
            pltpu.SMEM((1, 1), jnp.float32),
            pltpu.VMEM((1, D), jnp.float32),
            pltpu.VMEM((1, VPAD), jnp.float32),
        ],
    )(q2v_W, q2v_b2d, cnt, cw, bc2)


def kernel(input_document, emb_w, emb_concept_w, q2v_W, q2v_b):
    doc = input_document.astype(jnp.int32)
    zeros_v = jnp.zeros((V,), jnp.float32)
    ones_g = jnp.ones((GCHUNK,), jnp.float32)
    embedded, bc2 = _sc_gather_bincount(doc, emb_w, zeros_v, ones_g)
    enc, qw, cnt, vq, lts = _tc_vq(embedded, emb_concept_w)
    outs, m, s, docu = _tc_vocab(q2v_W, q2v_b.reshape(1, V), cnt,
                                 emb_concept_w, bc2)
    return (enc, qw, docu, outs, vq.reshape(()), lts.reshape(()))


# speedup vs baseline: 1.4168x; 1.0274x over previous
import functools

import jax
import jax.numpy as jnp
from jax import lax
from jax.experimental import pallas as pl
from jax.experimental.pallas import tpu as pltpu
from jax.experimental.pallas import tpu_sc as plsc

V = 50000
K = 512
D = 256
N = 32768

NC = 2
NS = 16
NW = NC * NS
TOK_PER_TILE = N // NW
GCHUNK = 128
NCHUNK = TOK_PER_TILE // GCHUNK


def _sc_gather_bincount(doc, emb_w, zeros_v, ones_g):
    mesh = plsc.VectorSubcoreMesh(core_axis_name="c", subcore_axis_name="s")

    @functools.partial(
        pl.kernel,
        mesh=mesh,
        out_type=(
            jax.ShapeDtypeStruct((N, D), jnp.float32),
            jax.ShapeDtypeStruct((NC, V), jnp.float32),
        ),
        scratch_types=[
            pltpu.VMEM((NCHUNK, GCHUNK), jnp.int32),
            pltpu.VMEM((2, GCHUNK, D), jnp.float32),
            pltpu.VMEM((GCHUNK,), jnp.float32),
            pltpu.VMEM_SHARED((V,), jnp.float32),
            pltpu.SemaphoreType.DMA((2,)),
            pltpu.SemaphoreType.DMA((2,)),
        ],
    )
    def k(doc_hbm, emb_hbm, zeros_hbm, ones_hbm, out_hbm, bc_hbm,
          idx_v, rows_v, ones_v, hist_sh, gsem, wsem):
        cid = lax.axis_index("c")
        sid = lax.axis_index("s")
        wid = sid * NC + cid
        base = wid * TOK_PER_TILE

        @pl.when(sid == 0)
        def _():
            pltpu.sync_copy(zeros_hbm, hist_sh)

        pltpu.sync_copy(ones_hbm, ones_v)
        for j in range(NCHUNK):
            pltpu.sync_copy(doc_hbm.at[pl.ds(base + j * GCHUNK, GCHUNK)],
                            idx_v.at[j])
        plsc.subcore_barrier()

        def gather(j):
            return pltpu.async_copy(emb_hbm.at[idx_v.at[j]],
                                    rows_v.at[j % 2], gsem.at[j % 2])

        def write(j):
            return pltpu.async_copy(rows_v.at[j % 2],
                                    out_hbm.at[pl.ds(base + j * GCHUNK,
                                                     GCHUNK)],
                                    wsem.at[j % 2])

        gathers = [None] * NCHUNK
        writes = [None] * NCHUNK
        gathers[0] = gather(0)
        for j in range(NCHUNK):
            if j >= 1:
                writes[j - 1].wait()
            if j + 1 < NCHUNK:
                gathers[j + 1] = gather(j + 1)
            gathers[j].wait()
            writes[j] = write(j)
            pltpu.sync_copy(ones_v, hist_sh.at[idx_v.at[j]], add=True)
        writes[NCHUNK - 1].wait()

        plsc.subcore_barrier()

        @pl.when(sid == 0)
        def _():
            pltpu.sync_copy(hist_sh, bc_hbm.at[cid])

    return k(doc, emb_w, zeros_v, ones_g)


BN = 4096
NB = N // BN


def _vq_body(e_ref, c_ref, enc_ref, qw_ref, cnt_ref, vq_ref, lts_ref,
             acc_ref, vqs_ref):
    i = pl.program_id(0)
    e = e_ref[...]
    c = c_ref[...]

    @pl.when(i == 0)
    def _():
        g = lax.dot_general(c, c, (((1,), (1,)), ((), ())))
        nrm = jnp.sum(c * c, axis=1)
        sm = jnp.sum(c, axis=1)
        d2 = (nrm[:, None] + nrm[None, :] - 2.0 * g
              + 2e-6 * (sm[:, None] - sm[None, :]) + D * 1e-12)
        dist = jnp.sqrt(jnp.maximum(d2, 0.0))
        r = lax.broadcasted_iota(jnp.int32, (K, K), 0)
        cc = lax.broadcasted_iota(jnp.int32, (K, K), 1)
        losses = jnp.where(r == cc, dist, jnp.maximum(0.0, 1.0 - dist))
        lts_ref[0, 0] = jnp.sum(losses) / (K * K)
    e2 = jnp.sum(e * e, axis=1, keepdims=True)
    c2 = jnp.sum(c * c, axis=1)
    cross = lax.dot_general(e, c, (((1,), (1,)), ((), ())))
    dist = e2 + c2[None, :] - 2.0 * cross
    m = jnp.min(dist, axis=1, keepdims=True)
    kiota = lax.broadcasted_iota(jnp.int32, (BN, K), 1)
    idx = jnp.min(jnp.where(dist == m, kiota, K), axis=1, keepdims=True)
    onehot = (kiota == idx).astype(jnp.float32)
    enc_ref[...] = onehot
    qw_ref[...] = jnp.dot(onehot, c)

    @pl.when(i == 0)
    def _():
        acc_ref[...] = jnp.zeros_like(acc_ref)
        vqs_ref[0, 0] = 0.0

    acc_ref[...] += jnp.sum(onehot, axis=0, keepdims=True)
    vqs_ref[0, 0] += jnp.sum(m)

    @pl.when(i == NB - 1)
    def _():
        cnt_ref[...] = acc_ref[...]
        mloss = vqs_ref[0, 0] / (N * D)
        vq_ref[0, 0] = mloss + 0.25 * mloss


def _tc_vq(embedded, cw):
    return pl.pallas_call(
        _vq_body,
        grid=(NB,),
        in_specs=[
            pl.BlockSpec((BN, D), lambda i: (i, 0)),
            pl.BlockSpec((K, D), lambda i: (0, 0)),
        ],
        out_specs=[
            pl.BlockSpec((BN, K), lambda i: (i, 0)),
            pl.BlockSpec((BN, D), lambda i: (i, 0)),
            pl.BlockSpec((1, K), lambda i: (0, 0)),
            pl.BlockSpec((1, 1), lambda i: (0, 0), memory_space=pltpu.SMEM),
            pl.BlockSpec((1, 1), lambda i: (0, 0), memory_space=pltpu.SMEM),
        ],
        out_shape=[
            jax.ShapeDtypeStruct((N, K), jnp.float32),
            jax.ShapeDtypeStruct((N, D), jnp.float32),
            jax.ShapeDtypeStruct((1, K), jnp.float32),
            jax.ShapeDtypeStruct((1, 1), jnp.float32),
            jax.ShapeDtypeStruct((1, 1), jnp.float32),
        ],
        scratch_shapes=[
            pltpu.VMEM((1, K), jnp.float32),
            pltpu.SMEM((1, 1), jnp.float32),
        ],
    )(embedded, cw)


BV = 2048
NVB = (V + BV - 1) // BV
VPAD = NVB * BV


def _vocab_body(w_ref, b_ref, cnt_ref, c_ref, bc_ref,
                out_ref, m_ref, s_ref, docu_ref,
                mm_ref, ss_ref, docu_v, lg_scr):
    p = pl.program_id(0)
    j = pl.program_id(1)

    @pl.when((p == 0) & (j == 0))
    def _():
        docu_v[...] = lax.dot_general(cnt_ref[...], c_ref[...],
                                      (((1,), (0,)), ((), ()))) / N
        docu_ref[...] = docu_v[...]
        mm_ref[0, 0] = -jnp.inf
        ss_ref[0, 0] = 0.0

    @pl.when(p == 0)
    def _():
        w = w_ref[...]
        docu = docu_v[...]
        lg = lax.dot_general(docu, w, (((1,), (1,)), ((), ()))) + b_ref[...]
        lg_scr[0:1, pl.ds(j * BV, BV)] = lg
        viota = lax.broadcasted_iota(jnp.int32, (1, BV), 1) + j * BV
        valid = viota < V
        lgv = jnp.where(valid, lg, -jnp.inf)
        bm = jnp.max(lgv)
        m_old = mm_ref[0, 0]
        m_new = jnp.maximum(m_old, bm)
        ssum = jnp.sum(jnp.where(valid, jnp.exp(lg - m_new), 0.0))
        ss_ref[0, 0] = ss_ref[0, 0] * jnp.exp(m_old - m_new) + ssum
        mm_ref[0, 0] = m_new

        @pl.when(j == NVB - 1)
        def _():
            m_ref[0, 0] = mm_ref[0, 0]
            s_ref[0, 0] = ss_ref[0, 0]

    @pl.when(p == 1)
    def _():
        lg = lg_scr[0:1, pl.ds(j * BV, BV)]
        smax = jnp.exp(lg - mm_ref[0, 0]) / ss_ref[0, 0]
        bc = jnp.sum(bc_ref[...], axis=0, keepdims=True)
        out_ref[...] = jnp.log(smax + 1e-6) * bc


def _tc_vocab(q2v_W, q2v_b2d, cnt, cw, bc2):
    return pl.pallas_call(
        _vocab_body,
        grid=(2, NVB),
        in_specs=[
            pl.BlockSpec((BV, D), lambda p, j: ((1 - p) * j, 0)),
            pl.BlockSpec((1, BV), lambda p, j: (0, (1 - p) * j)),
            pl.BlockSpec((1, K), lambda p, j: (0, 0)),
            pl.BlockSpec((K, D), lambda p, j: (0, 0)),
            pl.BlockSpec((NC, BV), lambda p, j: (0, p * j)),
        ],
        out_specs=[
            pl.BlockSpec((1, BV), lambda p, j: (0, p * j)),
            pl.BlockSpec((1, 1), lambda p, j: (0, 0), memory_space=pltpu.SMEM),
            pl.BlockSpec((1, 1), lambda p, j: (0, 0), memory_space=pltpu.SMEM),
            pl.BlockSpec((1, D), lambda p, j: (0, 0)),
        ],
        out_shape=[
            jax.ShapeDtypeStruct((1, V), jnp.float32),
            jax.ShapeDtypeStruct((1, 1), jnp.float32),
            jax.ShapeDtypeStruct((1, 1), jnp.float32),
            jax.ShapeDtypeStruct((1, D), jnp.float32),
        ],
        scratch_shapes=[
            pltpu.SMEM((1, 1), jnp.float32),
            pltpu.SMEM((1, 1), jnp.float32),
            pltpu.VMEM((1, D), jnp.float32),
            pltpu.VMEM((1, VPAD), jnp.float32),
        ],
    )(q2v_W, q2v_b2d, cnt, cw, bc2)


def kernel(input_document, emb_w, emb_concept_w, q2v_W, q2v_b):
    doc = input_document.astype(jnp.int32)
    zeros_v = jnp.zeros((V,), jnp.float32)
    ones_g = jnp.ones((GCHUNK,), jnp.float32)
    embedded, bc2 = _sc_gather_bincount(doc, emb_w, zeros_v, ones_g)
    enc, qw, cnt, vq, lts = _tc_vq(embedded, emb_concept_w)
    outs, m, s, docu = _tc_vocab(q2v_W, q2v_b.reshape(1, V), cnt,
                                 emb_concept_w, bc2)
    return (enc, qw, docu, outs, vq.reshape(()), lts.reshape(()))


# speedup vs baseline: 1.4179x; 1.0008x over previous
import functools

import jax
import jax.numpy as jnp
from jax import lax
from jax.experimental import pallas as pl
from jax.experimental.pallas import tpu as pltpu
from jax.experimental.pallas import tpu_sc as plsc

V = 50000
K = 512
D = 256
N = 32768

NC = 2
NS = 16
NW = NC * NS
TOK_PER_TILE = N // NW
GCHUNK = 128
NCHUNK = TOK_PER_TILE // GCHUNK


def _sc_gather_bincount(doc, emb_w, zeros_v, ones_g):
    mesh = plsc.VectorSubcoreMesh(core_axis_name="c", subcore_axis_name="s")

    @functools.partial(
        pl.kernel,
        mesh=mesh,
        out_type=(
            jax.ShapeDtypeStruct((N, D), jnp.float32),
            jax.ShapeDtypeStruct((NC, V), jnp.float32),
        ),
        scratch_types=[
            pltpu.VMEM((NCHUNK, GCHUNK), jnp.int32),
            pltpu.VMEM((2, GCHUNK, D), jnp.float32),
            pltpu.VMEM((GCHUNK,), jnp.float32),
            pltpu.VMEM_SHARED((V,), jnp.float32),
            pltpu.SemaphoreType.DMA((2,)),
            pltpu.SemaphoreType.DMA((2,)),
        ],
    )
    def k(doc_hbm, emb_hbm, zeros_hbm, ones_hbm, out_hbm, bc_hbm,
          idx_v, rows_v, ones_v, hist_sh, gsem, wsem):
        cid = lax.axis_index("c")
        sid = lax.axis_index("s")
        wid = sid * NC + cid
        base = wid * TOK_PER_TILE

        @pl.when(sid == 0)
        def _():
            pltpu.sync_copy(zeros_hbm, hist_sh)

        pltpu.sync_copy(ones_hbm, ones_v)
        for j in range(NCHUNK):
            pltpu.sync_copy(doc_hbm.at[pl.ds(base + j * GCHUNK, GCHUNK)],
                            idx_v.at[j])
        plsc.subcore_barrier()

        def gather(j):
            return pltpu.async_copy(emb_hbm.at[idx_v.at[j]],
                                    rows_v.at[j % 2], gsem.at[j % 2])

        def write(j):
            return pltpu.async_copy(rows_v.at[j % 2],
                                    out_hbm.at[pl.ds(base + j * GCHUNK,
                                                     GCHUNK)],
                                    wsem.at[j % 2])

        gathers = [None] * NCHUNK
        writes = [None] * NCHUNK
        gathers[0] = gather(0)
        for j in range(NCHUNK):
            if j >= 1:
                writes[j - 1].wait()
            if j + 1 < NCHUNK:
                gathers[j + 1] = gather(j + 1)
            gathers[j].wait()
            writes[j] = write(j)
            pltpu.sync_copy(ones_v, hist_sh.at[idx_v.at[j]], add=True)
        writes[NCHUNK - 1].wait()

        plsc.subcore_barrier()

        @pl.when(sid == 0)
        def _():
            pltpu.sync_copy(hist_sh, bc_hbm.at[cid])

    return k(doc, emb_w, zeros_v, ones_g)


BN = 4096
NB = N // BN


BV = 2048
NVB = (V + BV - 1) // BV
VPAD = NVB * BV


def _mega_body(e_ref, c_ref, w_ref, b_ref, bc_ref,
               enc_ref, qw_ref, docu_ref, out_ref, vq_ref, lts_ref,
               acc_ref, vqs_ref, mm_ref, ss_ref, docu_v, lg_scr):
    t = pl.program_id(0)

    @pl.when(t < NB)
    def _():
        _vq_step(t, e_ref, c_ref, enc_ref, qw_ref, vq_ref, lts_ref,
                 acc_ref, vqs_ref)

    @pl.when(t == NB)
    def _():
        docu_v[...] = lax.dot_general(acc_ref[...], c_ref[...],
                                      (((1,), (0,)), ((), ()))) / N
        docu_ref[...] = docu_v[...]
        mm_ref[0, 0] = -jnp.inf
        ss_ref[0, 0] = 0.0

    @pl.when((t >= NB) & (t < NB + NVB))
    def _():
        j = t - NB
        w = w_ref[...]
        docu = docu_v[...]
        lg = lax.dot_general(docu, w, (((1,), (1,)), ((), ()))) + b_ref[...]
        lg_scr[0:1, pl.ds(j * BV, BV)] = lg
        viota = lax.broadcasted_iota(jnp.int32, (1, BV), 1) + j * BV
        valid = viota < V
        lgv = jnp.where(valid, lg, -jnp.inf)
        bm = jnp.max(lgv)
        m_old = mm_ref[0, 0]
        m_new = jnp.maximum(m_old, bm)
        ssum = jnp.sum(jnp.where(valid, jnp.exp(lg - m_new), 0.0))
        ss_ref[0, 0] = ss_ref[0, 0] * jnp.exp(m_old - m_new) + ssum
        mm_ref[0, 0] = m_new

    @pl.when(t >= NB + NVB)
    def _():
        j = t - NB - NVB
        lg = lg_scr[0:1, pl.ds(j * BV, BV)]
        smax = jnp.exp(lg - mm_ref[0, 0]) / ss_ref[0, 0]
        bc = jnp.sum(bc_ref[...], axis=0, keepdims=True)
        out_ref[...] = jnp.log(smax + 1e-6) * bc


def _vq_step(i, e_ref, c_ref, enc_ref, qw_ref, vq_ref, lts_ref,
             acc_ref, vqs_ref):
    e = e_ref[...]
    c = c_ref[...]

    @pl.when(i == 0)
    def _():
        g = lax.dot_general(c, c, (((1,), (1,)), ((), ())))
        nrm = jnp.sum(c * c, axis=1)
        sm = jnp.sum(c, axis=1)
        d2 = (nrm[:, None] + nrm[None, :] - 2.0 * g
              + 2e-6 * (sm[:, None] - sm[None, :]) + D * 1e-12)
        dist = jnp.sqrt(jnp.maximum(d2, 0.0))
        r = lax.broadcasted_iota(jnp.int32, (K, K), 0)
        cc = lax.broadcasted_iota(jnp.int32, (K, K), 1)
        losses = jnp.where(r == cc, dist, jnp.maximum(0.0, 1.0 - dist))
        lts_ref[0, 0] = jnp.sum(losses) / (K * K)
    e2 = jnp.sum(e * e, axis=1, keepdims=True)
    c2 = jnp.sum(c * c, axis=1)
    cross = lax.dot_general(e, c, (((1,), (1,)), ((), ())))
    dist = e2 + c2[None, :] - 2.0 * cross
    m = jnp.min(dist, axis=1, keepdims=True)
    kiota = lax.broadcasted_iota(jnp.int32, (BN, K), 1)
    idx = jnp.min(jnp.where(dist == m, kiota, K), axis=1, keepdims=True)
    onehot = (kiota == idx).astype(jnp.float32)
    enc_ref[...] = onehot
    qw_ref[...] = jnp.dot(onehot, c)

    @pl.when(i == 0)
    def _():
        acc_ref[...] = jnp.zeros_like(acc_ref)
        vqs_ref[0, 0] = 0.0

    acc_ref[...] += jnp.sum(onehot, axis=0, keepdims=True)
    vqs_ref[0, 0] += jnp.sum(m)

    @pl.when(i == NB - 1)
    def _():
        mloss = vqs_ref[0, 0] / (N * D)
        vq_ref[0, 0] = mloss + 0.25 * mloss


def _tc_mega(embedded, cw, q2v_W, q2v_b2d, bc2):
    nb1 = NB - 1

    return pl.pallas_call(
        _mega_body,
        grid=(NB + 2 * NVB,),
        in_specs=[
            pl.BlockSpec((BN, D), lambda t: (jnp.minimum(t, nb1), 0)),
            pl.BlockSpec((K, D), lambda t: (0, 0)),
            pl.BlockSpec((BV, D),
                         lambda t: (jnp.clip(t - NB, 0, NVB - 1), 0)),
            pl.BlockSpec((1, BV),
                         lambda t: (0, jnp.clip(t - NB, 0, NVB - 1))),
            pl.BlockSpec((NC, BV),
                         lambda t: (0, jnp.clip(t - NB - NVB, 0, NVB - 1))),
        ],
        out_specs=[
            pl.BlockSpec((BN, K), lambda t: (jnp.minimum(t, nb1), 0)),
            pl.BlockSpec((BN, D), lambda t: (jnp.minimum(t, nb1), 0)),
            pl.BlockSpec((1, D), lambda t: (0, 0)),
            pl.BlockSpec((1, BV),
                         lambda t: (0, jnp.clip(t - NB - NVB, 0, NVB - 1))),
            pl.BlockSpec((1, 1), lambda t: (0, 0), memory_space=pltpu.SMEM),
            pl.BlockSpec((1, 1), lambda t: (0, 0), memory_space=pltpu.SMEM),
        ],
        out_shape=[
            jax.ShapeDtypeStruct((N, K), jnp.float32),
            jax.ShapeDtypeStruct((N, D), jnp.float32),
            jax.ShapeDtypeStruct((1, D), jnp.float32),
            jax.ShapeDtypeStruct((1, V), jnp.float32),
            jax.ShapeDtypeStruct((1, 1), jnp.float32),
            jax.ShapeDtypeStruct((1, 1), jnp.float32),
        ],
        scratch_shapes=[
            pltpu.VMEM((1, K), jnp.float32),
            pltpu.SMEM((1, 1), jnp.float32),
            pltpu.SMEM((1, 1), jnp.float32),
            pltpu.SMEM((1, 1), jnp.float32),
            pltpu.VMEM((1, D), jnp.float32),
            pltpu.VMEM((1, VPAD), jnp.float32),
        ],
    )(embedded, cw, q2v_W, q2v_b2d, bc2)


def kernel(input_document, emb_w, emb_concept_w, q2v_W, q2v_b):
    doc = input_document.astype(jnp.int32)
    zeros_v = jnp.zeros((V,), jnp.float32)
    ones_g = jnp.ones((GCHUNK,), jnp.float32)
    embedded, bc2 = _sc_gather_bincount(doc, emb_w, zeros_v, ones_g)
    enc, qw, docu, outs, vq, lts = _tc_mega(
        embedded, emb_concept_w, q2v_W, q2v_b.reshape(1, V), bc2)
    return (enc, qw, docu, outs, vq.reshape(()), lts.reshape(()))


# speedup vs baseline: 1.5949x; 1.1248x over previous
import functools

import jax
import jax.numpy as jnp
from jax import lax
from jax.experimental import pallas as pl
from jax.experimental.pallas import tpu as pltpu
from jax.experimental.pallas import tpu_sc as plsc

V = 50000
K = 512
D = 256
N = 32768

NC = 2
NS = 16
NW = NC * NS
TOK_PER_TILE = N // NW
GCHUNK = 128
NCHUNK = TOK_PER_TILE // GCHUNK


def _sc_gather_bincount(doc, emb_w, zeros_v, ones_g):
    mesh = plsc.VectorSubcoreMesh(core_axis_name="c", subcore_axis_name="s")

    @functools.partial(
        pl.kernel,
        mesh=mesh,
        out_type=(
            jax.ShapeDtypeStruct((N, D), jnp.float32),
            jax.ShapeDtypeStruct((NC, V), jnp.float32),
        ),
        scratch_types=[
            pltpu.VMEM((NCHUNK, GCHUNK), jnp.int32),
            pltpu.VMEM((2, GCHUNK, D), jnp.float32),
            pltpu.VMEM((GCHUNK,), jnp.float32),
            pltpu.VMEM_SHARED((V,), jnp.float32),
            pltpu.SemaphoreType.DMA((2,)),
            pltpu.SemaphoreType.DMA((2,)),
        ],
    )
    def k(doc_hbm, emb_hbm, zeros_hbm, ones_hbm, out_hbm, bc_hbm,
          idx_v, rows_v, ones_v, hist_sh, gsem, wsem):
        cid = lax.axis_index("c")
        sid = lax.axis_index("s")
        wid = sid * NC + cid
        base = wid * TOK_PER_TILE

        @pl.when(sid == 0)
        def _():
            pltpu.sync_copy(zeros_hbm, hist_sh)

        pltpu.sync_copy(ones_hbm, ones_v)
        pltpu.sync_copy(doc_hbm.at[wid], idx_v)
        plsc.subcore_barrier()

        def gather(j):
            return pltpu.async_copy(emb_hbm.at[idx_v.at[j]],
                                    rows_v.at[j % 2], gsem.at[j % 2])

        def write(j):
            return pltpu.async_copy(rows_v.at[j % 2],
                                    out_hbm.at[pl.ds(base + j * GCHUNK,
                                                     GCHUNK)],
                                    wsem.at[j % 2])

        gathers = [None] * NCHUNK
        writes = [None] * NCHUNK
        gathers[0] = gather(0)
        for j in range(NCHUNK):
            if j >= 1:
                writes[j - 1].wait()
            if j + 1 < NCHUNK:
                gathers[j + 1] = gather(j + 1)
            gathers[j].wait()
            writes[j] = write(j)
            pltpu.sync_copy(ones_v, hist_sh.at[idx_v.at[j]], add=True)
        writes[NCHUNK - 1].wait()

        plsc.subcore_barrier()

        @pl.when(sid == 0)
        def _():
            pltpu.sync_copy(hist_sh, bc_hbm.at[cid])

    return k(doc, emb_w, zeros_v, ones_g)


BN = 4096
NB = N // BN


BV = 4096
NVB = (V + BV - 1) // BV
VPAD = NVB * BV


def _mega_body(e_ref, c_ref, w_ref, b_ref, bc_ref,
               enc_ref, qw_ref, docu_ref, out_ref, vq_ref, lts_ref,
               acc_ref, vqs_ref, mm_ref, ss_ref, docu_v, lg_scr):
    t = pl.program_id(0)

    @pl.when(t < NB)
    def _():
        _vq_step(t, e_ref, c_ref, enc_ref, qw_ref, vq_ref, lts_ref,
                 acc_ref, vqs_ref)

    @pl.when(t == NB)
    def _():
        docu_v[...] = lax.dot_general(acc_ref[...], c_ref[...],
                                      (((1,), (0,)), ((), ()))) / N
        docu_ref[...] = docu_v[...]
        mm_ref[0, 0] = -jnp.inf
        ss_ref[0, 0] = 0.0

    @pl.when((t >= NB) & (t < NB + NVB))
    def _():
        j = t - NB
        w = w_ref[...]
        docu = docu_v[...]
        lg = lax.dot_general(docu, w, (((1,), (1,)), ((), ()))) + b_ref[...]
        lg_scr[0:1, pl.ds(j * BV, BV)] = lg
        viota = lax.broadcasted_iota(jnp.int32, (1, BV), 1) + j * BV
        valid = viota < V
        lgv = jnp.where(valid, lg, -jnp.inf)
        bm = jnp.max(lgv)
        m_old = mm_ref[0, 0]
        m_new = jnp.maximum(m_old, bm)
        ssum = jnp.sum(jnp.where(valid, jnp.exp(lg - m_new), 0.0))
        ss_ref[0, 0] = ss_ref[0, 0] * jnp.exp(m_old - m_new) + ssum
        mm_ref[0, 0] = m_new

    @pl.when(t >= NB + NVB)
    def _():
        j = t - NB - NVB
        lg = lg_scr[0:1, pl.ds(j * BV, BV)]
        smax = jnp.exp(lg - mm_ref[0, 0]) / ss_ref[0, 0]
        bc = jnp.sum(bc_ref[...], axis=0, keepdims=True)
        out_ref[...] = jnp.log(smax + 1e-6) * bc


def _vq_step(i, e_ref, c_ref, enc_ref, qw_ref, vq_ref, lts_ref,
             acc_ref, vqs_ref):
    e = e_ref[...]
    c = c_ref[...]

    @pl.when(i == 0)
    def _():
        g = lax.dot_general(c, c, (((1,), (1,)), ((), ())))
        nrm = jnp.sum(c * c, axis=1)
        sm = jnp.sum(c, axis=1)
        d2 = (nrm[:, None] + nrm[None, :] - 2.0 * g
              + 2e-6 * (sm[:, None] - sm[None, :]) + D * 1e-12)
        dist = jnp.sqrt(jnp.maximum(d2, 0.0))
        r = lax.broadcasted_iota(jnp.int32, (K, K), 0)
        cc = lax.broadcasted_iota(jnp.int32, (K, K), 1)
        losses = jnp.where(r == cc, dist, jnp.maximum(0.0, 1.0 - dist))
        lts_ref[0, 0] = jnp.sum(losses) / (K * K)
    e2 = jnp.sum(e * e, axis=1, keepdims=True)
    c2 = jnp.sum(c * c, axis=1)
    cross = lax.dot_general(e, c, (((1,), (1,)), ((), ())))
    dist = e2 + c2[None, :] - 2.0 * cross
    m = jnp.min(dist, axis=1, keepdims=True)
    kiota = lax.broadcasted_iota(jnp.int32, (BN, K), 1)
    idx = jnp.min(jnp.where(dist == m, kiota, K), axis=1, keepdims=True)
    onehot = (kiota == idx).astype(jnp.float32)
    enc_ref[...] = onehot
    qw_ref[...] = jnp.dot(onehot, c)

    @pl.when(i == 0)
    def _():
        acc_ref[...] = jnp.zeros_like(acc_ref)
        vqs_ref[0, 0] = 0.0

    acc_ref[...] += jnp.sum(onehot, axis=0, keepdims=True)
    vqs_ref[0, 0] += jnp.sum(m)

    @pl.when(i == NB - 1)
    def _():
        mloss = vqs_ref[0, 0] / (N * D)
        vq_ref[0, 0] = mloss + 0.25 * mloss


def _tc_mega(embedded, cw, q2v_W, q2v_b2d, bc2):
    nb1 = NB - 1

    return pl.pallas_call(
        _mega_body,
        grid=(NB + 2 * NVB,),
        in_specs=[
            pl.BlockSpec((BN, D), lambda t: (jnp.minimum(t, nb1), 0)),
            pl.BlockSpec((K, D), lambda t: (0, 0)),
            pl.BlockSpec((BV, D),
                         lambda t: (jnp.clip(t - NB, 0, NVB - 1), 0)),
            pl.BlockSpec((1, BV),
                         lambda t: (0, jnp.clip(t - NB, 0, NVB - 1))),
            pl.BlockSpec((NC, BV),
                         lambda t: (0, jnp.clip(t - NB - NVB, 0, NVB - 1))),
        ],
        out_specs=[
            pl.BlockSpec((BN, K), lambda t: (jnp.minimum(t, nb1), 0)),
            pl.BlockSpec((BN, D), lambda t: (jnp.minimum(t, nb1), 0)),
            pl.BlockSpec((1, D), lambda t: (0, 0)),
            pl.BlockSpec((1, BV),
                         lambda t: (0, jnp.clip(t - NB - NVB, 0, NVB - 1))),
            pl.BlockSpec((1, 1), lambda t: (0, 0), memory_space=pltpu.SMEM),
            pl.BlockSpec((1, 1), lambda t: (0, 0), memory_space=pltpu.SMEM),
        ],
        out_shape=[
            jax.ShapeDtypeStruct((N, K), jnp.float32),
            jax.ShapeDtypeStruct((N, D), jnp.float32),
            jax.ShapeDtypeStruct((1, D), jnp.float32),
            jax.ShapeDtypeStruct((1, V), jnp.float32),
            jax.ShapeDtypeStruct((1, 1), jnp.float32),
            jax.ShapeDtypeStruct((1, 1), jnp.float32),
        ],
        scratch_shapes=[
            pltpu.VMEM((1, K), jnp.float32),
            pltpu.SMEM((1, 1), jnp.float32),
            pltpu.SMEM((1, 1), jnp.float32),
            pltpu.SMEM((1, 1), jnp.float32),
            pltpu.VMEM((1, D), jnp.float32),
            pltpu.VMEM((1, VPAD), jnp.float32),
        ],
    )(embedded, cw, q2v_W, q2v_b2d, bc2)


def kernel(input_document, emb_w, emb_concept_w, q2v_W, q2v_b):
    doc = input_document.astype(jnp.int32)
    zeros_v = jnp.zeros((V,), jnp.float32)
    ones_g = jnp.ones((GCHUNK,), jnp.float32)
    embedded, bc2 = _sc_gather_bincount(doc.reshape(NW, NCHUNK, GCHUNK),
                                        emb_w, zeros_v, ones_g)
    enc, qw, docu, outs, vq, lts = _tc_mega(
        embedded, emb_concept_w, q2v_W, q2v_b.reshape(1, V), bc2)
    return (enc, qw, docu, outs, vq.reshape(()), lts.reshape(()))


# speedup vs baseline: 1.6079x; 1.0081x over previous
import functools

import jax
import jax.numpy as jnp
from jax import lax
from jax.experimental import pallas as pl
from jax.experimental.pallas import tpu as pltpu
from jax.experimental.pallas import tpu_sc as plsc

V = 50000
K = 512
D = 256
N = 32768

NC = 2
NS = 16
NW = NC * NS
TOK_PER_TILE = N // NW
GCHUNK = 128
NCHUNK = TOK_PER_TILE // GCHUNK


def _sc_gather_bincount(doc, emb_w, zeros_v, ones_g):
    mesh = plsc.VectorSubcoreMesh(core_axis_name="c", subcore_axis_name="s")

    @functools.partial(
        pl.kernel,
        mesh=mesh,
        out_type=(
            jax.ShapeDtypeStruct((N, D), jnp.float32),
            jax.ShapeDtypeStruct((NC, V), jnp.float32),
        ),
        scratch_types=[
            pltpu.VMEM((NCHUNK, GCHUNK), jnp.int32),
            pltpu.VMEM((3, GCHUNK, D), jnp.float32),
            pltpu.VMEM((GCHUNK,), jnp.float32),
            pltpu.VMEM_SHARED((V,), jnp.float32),
            pltpu.SemaphoreType.DMA((3,)),
            pltpu.SemaphoreType.DMA((3,)),
        ],
    )
    def k(doc_hbm, emb_hbm, zeros_hbm, ones_hbm, out_hbm, bc_hbm,
          idx_v, rows_v, ones_v, hist_sh, gsem, wsem):
        cid = lax.axis_index("c")
        sid = lax.axis_index("s")
        wid = sid * NC + cid
        base = wid * TOK_PER_TILE

        @pl.when(sid == 0)
        def _():
            pltpu.sync_copy(zeros_hbm, hist_sh)

        pltpu.sync_copy(ones_hbm, ones_v)
        pltpu.sync_copy(doc_hbm.at[wid], idx_v)
        plsc.subcore_barrier()

        def gather(j):
            return pltpu.async_copy(emb_hbm.at[idx_v.at[j]],
                                    rows_v.at[j % 3], gsem.at[j % 3])

        def write(j):
            return pltpu.async_copy(rows_v.at[j % 3],
                                    out_hbm.at[pl.ds(base + j * GCHUNK,
                                                     GCHUNK)],
                                    wsem.at[j % 3])

        gathers = [None] * NCHUNK
        writes = [None] * NCHUNK
        gathers[0] = gather(0)
        gathers[1] = gather(1)
        for j in range(NCHUNK):
            if j >= 2:
                writes[j - 2].wait()
            if j + 2 < NCHUNK:
                gathers[j + 2] = gather(j + 2)
            gathers[j].wait()
            writes[j] = write(j)
            pltpu.sync_copy(ones_v, hist_sh.at[idx_v.at[j]], add=True)
        writes[NCHUNK - 2].wait()
        writes[NCHUNK - 1].wait()

        plsc.subcore_barrier()

        @pl.when(sid == 0)
        def _():
            pltpu.sync_copy(hist_sh, bc_hbm.at[cid])

    return k(doc, emb_w, zeros_v, ones_g)


BN = 4096
NB = N // BN


BV = 4096
NVB = (V + BV - 1) // BV
VPAD = NVB * BV


def _mega_body(e_ref, c_ref, w_ref, b_ref, bc_ref,
               enc_ref, qw_ref, docu_ref, out_ref, vq_ref, lts_ref,
               acc_ref, vqs_ref, mm_ref, ss_ref, docu_v, lg_scr):
    t = pl.program_id(0)

    @pl.when(t < NB)
    def _():
        _vq_step(t, e_ref, c_ref, enc_ref, qw_ref, vq_ref, lts_ref,
                 acc_ref, vqs_ref)

    @pl.when(t == NB)
    def _():
        docu_v[...] = lax.dot_general(acc_ref[...], c_ref[...],
                                      (((1,), (0,)), ((), ()))) / N
        docu_ref[...] = docu_v[...]
        mm_ref[0, 0] = -jnp.inf
        ss_ref[0, 0] = 0.0

    @pl.when((t >= NB) & (t < NB + NVB))
    def _():
        j = t - NB
        w = w_ref[...]
        docu = docu_v[...]
        lg = lax.dot_general(docu, w, (((1,), (1,)), ((), ()))) + b_ref[...]
        lg_scr[0:1, pl.ds(j * BV, BV)] = lg
        viota = lax.broadcasted_iota(jnp.int32, (1, BV), 1) + j * BV
        valid = viota < V
        lgv = jnp.where(valid, lg, -jnp.inf)
        bm = jnp.max(lgv)
        m_old = mm_ref[0, 0]
        m_new = jnp.maximum(m_old, bm)
        ssum = jnp.sum(jnp.where(valid, jnp.exp(lg - m_new), 0.0))
        ss_ref[0, 0] = ss_ref[0, 0] * jnp.exp(m_old - m_new) + ssum
        mm_ref[0, 0] = m_new

    @pl.when(t >= NB + NVB)
    def _():
        j = t - NB - NVB
        lg = lg_scr[0:1, pl.ds(j * BV, BV)]
        smax = jnp.exp(lg - mm_ref[0, 0]) / ss_ref[0, 0]
        bc = jnp.sum(bc_ref[...], axis=0, keepdims=True)
        out_ref[...] = jnp.log(smax + 1e-6) * bc


def _vq_step(i, e_ref, c_ref, enc_ref, qw_ref, vq_ref, lts_ref,
             acc_ref, vqs_ref):
    e = e_ref[...]
    c = c_ref[...]

    @pl.when(i == 0)
    def _():
        g = lax.dot_general(c, c, (((1,), (1,)), ((), ())))
        nrm = jnp.sum(c * c, axis=1)
        sm = jnp.sum(c, axis=1)
        d2 = (nrm[:, None] + nrm[None, :] - 2.0 * g
              + 2e-6 * (sm[:, None] - sm[None, :]) + D * 1e-12)
        dist = jnp.sqrt(jnp.maximum(d2, 0.0))
        r = lax.broadcasted_iota(jnp.int32, (K, K), 0)
        cc = lax.broadcasted_iota(jnp.int32, (K, K), 1)
        losses = jnp.where(r == cc, dist, jnp.maximum(0.0, 1.0 - dist))
        lts_ref[0, 0] = jnp.sum(losses) / (K * K)
    e2 = jnp.sum(e * e, axis=1, keepdims=True)
    c2 = jnp.sum(c * c, axis=1)
    cross = lax.dot_general(e, c, (((1,), (1,)), ((), ())))
    dist = e2 + c2[None, :] - 2.0 * cross
    m = jnp.min(dist, axis=1, keepdims=True)
    kiota = lax.broadcasted_iota(jnp.int32, (BN, K), 1)
    idx = jnp.min(jnp.where(dist == m, kiota, K), axis=1, keepdims=True)
    onehot = (kiota == idx).astype(jnp.float32)
    enc_ref[...] = onehot
    qw_ref[...] = jnp.dot(onehot, c)

    @pl.when(i == 0)
    def _():
        acc_ref[...] = jnp.zeros_like(acc_ref)
        vqs_ref[0, 0] = 0.0

    acc_ref[...] += jnp.sum(onehot, axis=0, keepdims=True)
    vqs_ref[0, 0] += jnp.sum(m)

    @pl.when(i == NB - 1)
    def _():
        mloss = vqs_ref[0, 0] / (N * D)
        vq_ref[0, 0] = mloss + 0.25 * mloss


def _tc_mega(embedded, cw, q2v_W, q2v_b2d, bc2):
    nb1 = NB - 1

    return pl.pallas_call(
        _mega_body,
        grid=(NB + 2 * NVB,),
        in_specs=[
            pl.BlockSpec((BN, D), lambda t: (jnp.minimum(t, nb1), 0)),
            pl.BlockSpec((K, D), lambda t: (0, 0)),
            pl.BlockSpec((BV, D),
                         lambda t: (jnp.clip(t - NB, 0, NVB - 1), 0)),
            pl.BlockSpec((1, BV),
                         lambda t: (0, jnp.clip(t - NB, 0, NVB - 1))),
            pl.BlockSpec((NC, BV),
                         lambda t: (0, jnp.clip(t - NB - NVB, 0, NVB - 1))),
        ],
        out_specs=[
            pl.BlockSpec((BN, K), lambda t: (jnp.minimum(t, nb1), 0)),
            pl.BlockSpec((BN, D), lambda t: (jnp.minimum(t, nb1), 0)),
            pl.BlockSpec((1, D), lambda t: (0, 0)),
            pl.BlockSpec((1, BV),
                         lambda t: (0, jnp.clip(t - NB - NVB, 0, NVB - 1))),
            pl.BlockSpec((1, 1), lambda t: (0, 0), memory_space=pltpu.SMEM),
            pl.BlockSpec((1, 1), lambda t: (0, 0), memory_space=pltpu.SMEM),
        ],
        out_shape=[
            jax.ShapeDtypeStruct((N, K), jnp.float32),
            jax.ShapeDtypeStruct((N, D), jnp.float32),
            jax.ShapeDtypeStruct((1, D), jnp.float32),
            jax.ShapeDtypeStruct((1, V), jnp.float32),
            jax.ShapeDtypeStruct((1, 1), jnp.float32),
            jax.ShapeDtypeStruct((1, 1), jnp.float32),
        ],
        scratch_shapes=[
            pltpu.VMEM((1, K), jnp.float32),
            pltpu.SMEM((1, 1), jnp.float32),
            pltpu.SMEM((1, 1), jnp.float32),
            pltpu.SMEM((1, 1), jnp.float32),
            pltpu.VMEM((1, D), jnp.float32),
            pltpu.VMEM((1, VPAD), jnp.float32),
        ],
    )(embedded, cw, q2v_W, q2v_b2d, bc2)


def kernel(input_document, emb_w, emb_concept_w, q2v_W, q2v_b):
    doc = input_document.astype(jnp.int32)
    zeros_v = jnp.zeros((V,), jnp.float32)
    ones_g = jnp.ones((GCHUNK,), jnp.float32)
    embedded, bc2 = _sc_gather_bincount(doc.reshape(NW, NCHUNK, GCHUNK),
                                        emb_w, zeros_v, ones_g)
    enc, qw, docu, outs, vq, lts = _tc_mega(
        embedded, emb_concept_w, q2v_W, q2v_b.reshape(1, V), bc2)
    return (enc, qw, docu, outs, vq.reshape(()), lts.reshape(()))
